# baseline (speedup 1.0000x reference)
import functools

import jax
import jax.numpy as jnp
from jax import lax
from jax.experimental import pallas as pl
from jax.experimental.pallas import tpu as pltpu
from jax.experimental.pallas import tpu_sc as plsc

B, T, V, D, H = 1024, 200, 100000, 128, 64
CHUNKS = (40, 40, 40, 40, 40)
TSUB = 10
CH = 80
NBUF = 4


def _sc_gather_chunk(idx3d, table, tc):
    info = plsc.get_sparse_core_info()
    nw = info.num_cores * info.num_subcores
    per_w = (tc * B) // nw
    n_ch = per_w // CH
    mesh = plsc.VectorSubcoreMesh(core_axis_name="c", subcore_axis_name="s")

    @functools.partial(
        pl.kernel,
        mesh=mesh,
        out_type=jax.ShapeDtypeStruct((tc * B, D), jnp.float32),
        scratch_types=[
            pltpu.VMEM((n_ch, CH), jnp.int32),
            pltpu.VMEM((NBUF, CH, D), jnp.float32),
        ] + [pltpu.SemaphoreType.DMA] * (2 * NBUF),
    )
    def gather_kernel(idx_hbm, table_hbm, out_hbm, idx_v, bufs, *sems):
        gsem, wsem = sems[:NBUF], sems[NBUF:]
        wid = lax.axis_index("s") * info.num_cores + lax.axis_index("c")
        r0 = wid * per_w
        pltpu.sync_copy(idx_hbm.at[wid], idx_v)

        def g_copy(j):
            r = j % NBUF
            return pltpu.make_async_copy(
                table_hbm.at[idx_v.at[j]], bufs.at[r], gsem[r])

        def w_copy(j):
            r = j % NBUF
            return pltpu.make_async_copy(
                bufs.at[r], out_hbm.at[pl.ds(r0 + j * CH, CH)], wsem[r])

        g_copy(0).start()
        g_copy(1).start()
        for j in range(n_ch):
            g_copy(j).wait()
            w_copy(j).start()
            if j + 2 < n_ch:
                if j >= 2:
                    w_copy(j - 2).wait()
                g_copy(j + 2).start()
        for j in range(max(n_ch - 4, 2), n_ch):
            w_copy(j).wait()

    return gather_kernel(idx3d, table)


def _make_lstm_body(ngc, aliased):
    def lstm_body(emb_ref, h0_ref, c0_ref, wt_ref, ut_ref, b_ref,
                  out_ref, ht_ref, ct_ref, h_s, c_s):
        g = pl.program_id(0)

        @pl.when(g == 0)
        def _():
            h_s[...] = h0_ref[...]
            c_s[...] = c0_ref[...]

        ut = ut_ref[...]
        wt = wt_ref[...]
        bb = b_ref[...]
        b2 = B // 2
        hs = [h_s[:, 0:b2], h_s[:, b2:B]]
        cs = [c_s[:, 0:b2], c_s[:, b2:B]]
        nt = (((1,), (1,)), ((), ()))
        for j in range(TSUB):
            for p in range(2):
                x = emb_ref[j, p * b2:(p + 1) * b2, :]
                z = (lax.dot_general(wt, x, nt,
                                     preferred_element_type=jnp.float32)
                     + jnp.dot(ut, hs[p], preferred_element_type=jnp.float32)
                     + bb)
                s_if = jax.nn.sigmoid(z[:2 * H, :])
                i = s_if[:H, :]
                f = s_if[H:, :]
                gg = jnp.tanh(z[2 * H:3 * H, :])
                o = jax.nn.sigmoid(z[3 * H:, :])
                cs[p] = f * cs[p] + i * gg
                hs[p] = o * jnp.tanh(cs[p])
                out_ref[j * H:(j + 1) * H, p * b2:(p + 1) * b2] = hs[p]
        h_s[:, 0:b2] = hs[0]
        h_s[:, b2:B] = hs[1]
        c_s[:, 0:b2] = cs[0]
        c_s[:, b2:B] = cs[1]

        @pl.when(g == ngc - 1)
        def _():
            ht_ref[:, 0:b2] = hs[0]
            ht_ref[:, b2:B] = hs[1]
            ct_ref[:, 0:b2] = cs[0]
            ct_ref[:, b2:B] = cs[1]

    if not aliased:
        return lstm_body

    def lstm_body_aliased(emb_ref, h0_ref, c0_ref, wt_ref, ut_ref, b_ref,
                          obuf_ref, out_ref, ht_ref, ct_ref, h_s, c_s):
        del obuf_ref
        return lstm_body(emb_ref, h0_ref, c0_ref, wt_ref, ut_ref, b_ref,
                         out_ref, ht_ref, ct_ref, h_s, c_s)

    return lstm_body_aliased


def _tc_lstm_chunk(t0, tc, emb_k, ht, ct, wt, ut, b2d, out_buf):
    ngc = tc // TSUB
    in_specs = [
        pl.BlockSpec((TSUB, B, D), lambda g: (g, 0, 0)),
        pl.BlockSpec((H, B), lambda g: (0, 0)),
        pl.BlockSpec((H, B), lambda g: (0, 0)),
        pl.BlockSpec((4 * H, D), lambda g: (0, 0)),
        pl.BlockSpec((4 * H, H), lambda g: (0, 0)),
        pl.BlockSpec((4 * H, 1), lambda g: (0, 0)),
    ]
    operands = [emb_k, ht, ct, wt, ut, b2d]
    aliases = {}
    if out_buf is not None:
        in_specs.append(pl.BlockSpec(memory_space=pl.ANY))
        operands.append(out_buf)
        aliases = {6: 0}
    g0 = t0 // TSUB
    return pl.pallas_call(
        _make_lstm_body(ngc, out_buf is not None),
        grid=(ngc,),
        in_specs=in_specs,
        out_specs=[
            pl.BlockSpec((TSUB * H, B), lambda g: (g0 + g, 0)),
            pl.BlockSpec((H, B), lambda g: (0, 0)),
            pl.BlockSpec((H, B), lambda g: (0, 0)),
        ],
        out_shape=[
            jax.ShapeDtypeStruct((T * H, B), jnp.float32),
            jax.ShapeDtypeStruct((H, B), jnp.float32),
            jax.ShapeDtypeStruct((H, B), jnp.float32),
        ],
        scratch_shapes=[
            pltpu.VMEM((H, B), jnp.float32),
            pltpu.VMEM((H, B), jnp.float32),
        ],
        input_output_aliases=aliases,
    )(*operands)


def kernel(sequence, states_1, states_2, table, W, U, b):
    wt = jnp.transpose(W)
    ut = jnp.transpose(U)
    bt = b.reshape(4 * H, 1)
    ht = jnp.transpose(states_1)
    ct = jnp.transpose(states_2)
    out_buf = None
    seq_t = jnp.transpose(sequence)
    t0 = 0
    for tc in CHUNKS:
        idx3d = lax.slice_in_dim(seq_t, t0, t0 + tc, axis=0
                                 ).reshape(32, tc * B // (32 * CH), CH)
        emb_k = _sc_gather_chunk(idx3d, table, tc).reshape(tc, B, D)
        out_buf, ht, ct = _tc_lstm_chunk(t0, tc, emb_k, ht, ct, wt, ut, bt,
                                         out_buf)
        t0 += tc
    out = jnp.transpose(out_buf.reshape(T, H, B), (2, 0, 1))
    return out, jnp.transpose(ht), jnp.transpose(ct)


# speedup vs baseline: 1.0135x; 1.0135x over previous
import functools

import jax
import jax.numpy as jnp
from jax import lax
from jax.experimental import pallas as pl
from jax.experimental.pallas import tpu as pltpu
from jax.experimental.pallas import tpu_sc as plsc

B, T, V, D, H = 1024, 200, 100000, 128, 64
CHUNKS = (50, 50, 50, 50)
TSUB = 10
CH = 80
NBUF = 4


def _sc_gather_chunk(idx3d, table, tc):
    info = plsc.get_sparse_core_info()
    nw = info.num_cores * info.num_subcores
    per_w = (tc * B) // nw
    n_ch = per_w // CH
    mesh = plsc.VectorSubcoreMesh(core_axis_name="c", subcore_axis_name="s")

    @functools.partial(
        pl.kernel,
        mesh=mesh,
        out_type=jax.ShapeDtypeStruct((tc * B, D), jnp.float32),
        scratch_types=[
            pltpu.VMEM((n_ch, CH), jnp.int32),
            pltpu.VMEM((NBUF, CH, D), jnp.float32),
        ] + [pltpu.SemaphoreType.DMA] * (2 * NBUF),
    )
    def gather_kernel(idx_hbm, table_hbm, out_hbm, idx_v, bufs, *sems):
        gsem, wsem = sems[:NBUF], sems[NBUF:]
        wid = lax.axis_index("s") * info.num_cores + lax.axis_index("c")
        r0 = wid * per_w
        pltpu.sync_copy(idx_hbm.at[wid], idx_v)

        def g_copy(j):
            r = j % NBUF
            return pltpu.make_async_copy(
                table_hbm.at[idx_v.at[j]], bufs.at[r], gsem[r])

        def w_copy(j):
            r = j % NBUF
            return pltpu.make_async_copy(
                bufs.at[r], out_hbm.at[pl.ds(r0 + j * CH, CH)], wsem[r])

        g_copy(0).start()
        g_copy(1).start()
        for j in range(n_ch):
            g_copy(j).wait()
            w_copy(j).start()
            if j + 2 < n_ch:
                if j >= 2:
                    w_copy(j - 2).wait()
                g_copy(j + 2).start()
        for j in range(max(n_ch - 4, 2), n_ch):
            w_copy(j).wait()

    return gather_kernel(idx3d, table)


def _make_lstm_body(ngc, aliased):
    def lstm_body(emb_ref, h0_ref, c0_ref, wt_ref, ut_ref, b_ref,
                  out_ref, ht_ref, ct_ref, h_s, c_s):
        g = pl.program_id(0)

        @pl.when(g == 0)
        def _():
            h_s[...] = h0_ref[...]
            c_s[...] = c0_ref[...]

        ut = ut_ref[...]
        wt = wt_ref[...]
        bb = b_ref[...]
        b2 = B // 2
        hs = [h_s[:, 0:b2], h_s[:, b2:B]]
        cs = [c_s[:, 0:b2], c_s[:, b2:B]]
        nt = (((1,), (1,)), ((), ()))
        for j in range(TSUB):
            for p in range(2):
                x = emb_ref[j, p * b2:(p + 1) * b2, :]
                z = (lax.dot_general(wt, x, nt,
                                     preferred_element_type=jnp.float32)
                     + jnp.dot(ut, hs[p], preferred_element_type=jnp.float32)
                     + bb)
                s_if = jax.nn.sigmoid(z[:2 * H, :])
                i = s_if[:H, :]
                f = s_if[H:, :]
                gg = jnp.tanh(z[2 * H:3 * H, :])
                o = jax.nn.sigmoid(z[3 * H:, :])
                cs[p] = f * cs[p] + i * gg
                hs[p] = o * jnp.tanh(cs[p])
                out_ref[j * H:(j + 1) * H, p * b2:(p + 1) * b2] = hs[p]
        h_s[:, 0:b2] = hs[0]
        h_s[:, b2:B] = hs[1]
        c_s[:, 0:b2] = cs[0]
        c_s[:, b2:B] = cs[1]

        @pl.when(g == ngc - 1)
        def _():
            ht_ref[:, 0:b2] = hs[0]
            ht_ref[:, b2:B] = hs[1]
            ct_ref[:, 0:b2] = cs[0]
            ct_ref[:, b2:B] = cs[1]

    if not aliased:
        return lstm_body

    def lstm_body_aliased(emb_ref, h0_ref, c0_ref, wt_ref, ut_ref, b_ref,
                          obuf_ref, out_ref, ht_ref, ct_ref, h_s, c_s):
        del obuf_ref
        return lstm_body(emb_ref, h0_ref, c0_ref, wt_ref, ut_ref, b_ref,
                         out_ref, ht_ref, ct_ref, h_s, c_s)

    return lstm_body_aliased


def _tc_lstm_chunk(t0, tc, emb_k, ht, ct, wt, ut, b2d, out_buf):
    ngc = tc // TSUB
    in_specs = [
        pl.BlockSpec((TSUB, B, D), lambda g: (g, 0, 0)),
        pl.BlockSpec((H, B), lambda g: (0, 0)),
        pl.BlockSpec((H, B), lambda g: (0, 0)),
        pl.BlockSpec((4 * H, D), lambda g: (0, 0)),
        pl.BlockSpec((4 * H, H), lambda g: (0, 0)),
        pl.BlockSpec((4 * H, 1), lambda g: (0, 0)),
    ]
    operands = [emb_k, ht, ct, wt, ut, b2d]
    aliases = {}
    if out_buf is not None:
        in_specs.append(pl.BlockSpec(memory_space=pl.ANY))
        operands.append(out_buf)
        aliases = {6: 0}
    g0 = t0 // TSUB
    return pl.pallas_call(
        _make_lstm_body(ngc, out_buf is not None),
        grid=(ngc,),
        in_specs=in_specs,
        out_specs=[
            pl.BlockSpec((TSUB * H, B), lambda g: (g0 + g, 0)),
            pl.BlockSpec((H, B), lambda g: (0, 0)),
            pl.BlockSpec((H, B), lambda g: (0, 0)),
        ],
        out_shape=[
            jax.ShapeDtypeStruct((T * H, B), jnp.float32),
            jax.ShapeDtypeStruct((H, B), jnp.float32),
            jax.ShapeDtypeStruct((H, B), jnp.float32),
        ],
        scratch_shapes=[
            pltpu.VMEM((H, B), jnp.float32),
            pltpu.VMEM((H, B), jnp.float32),
        ],
        input_output_aliases=aliases,
    )(*operands)


def kernel(sequence, states_1, states_2, table, W, U, b):
    wt = jnp.transpose(W)
    ut = jnp.transpose(U)
    bt = b.reshape(4 * H, 1)
    ht = jnp.transpose(states_1)
    ct = jnp.transpose(states_2)
    out_buf = None
    seq_t = jnp.transpose(sequence)
    t0 = 0
    for tc in CHUNKS:
        idx3d = lax.slice_in_dim(seq_t, t0, t0 + tc, axis=0
                                 ).reshape(32, tc * B // (32 * CH), CH)
        emb_k = _sc_gather_chunk(idx3d, table, tc).reshape(tc, B, D)
        out_buf, ht, ct = _tc_lstm_chunk(t0, tc, emb_k, ht, ct, wt, ut, bt,
                                         out_buf)
        t0 += tc
    out = jnp.transpose(out_buf.reshape(T, H, B), (2, 0, 1))
    return out, jnp.transpose(ht), jnp.transpose(ct)


# speedup vs baseline: 1.0442x; 1.0304x over previous
import functools

import jax
import jax.numpy as jnp
from jax import lax
from jax.experimental import pallas as pl
from jax.experimental.pallas import tpu as pltpu
from jax.experimental.pallas import tpu_sc as plsc

B, T, V, D, H = 1024, 200, 100000, 128, 64
CHUNKS = (50, 50, 50, 50)
TSUB = 10
CH = 80
NBUF = 8
DEPTH = NBUF // 2


def _sc_gather_chunk(idx3d, table, tc):
    info = plsc.get_sparse_core_info()
    nw = info.num_cores * info.num_subcores
    per_w = (tc * B) // nw
    n_ch = per_w // CH
    mesh = plsc.VectorSubcoreMesh(core_axis_name="c", subcore_axis_name="s")

    @functools.partial(
        pl.kernel,
        mesh=mesh,
        out_type=jax.ShapeDtypeStruct((tc * B, D), jnp.float32),
        scratch_types=[
            pltpu.VMEM((n_ch, CH), jnp.int32),
            pltpu.VMEM((NBUF, CH, D), jnp.float32),
        ] + [pltpu.SemaphoreType.DMA] * (2 * NBUF),
    )
    def gather_kernel(idx_hbm, table_hbm, out_hbm, idx_v, bufs, *sems):
        gsem, wsem = sems[:NBUF], sems[NBUF:]
        wid = lax.axis_index("s") * info.num_cores + lax.axis_index("c")
        r0 = wid * per_w
        pltpu.sync_copy(idx_hbm.at[wid], idx_v)

        def g_copy(j):
            r = j % NBUF
            return pltpu.make_async_copy(
                table_hbm.at[idx_v.at[j]], bufs.at[r], gsem[r])

        def w_copy(j):
            r = j % NBUF
            return pltpu.make_async_copy(
                bufs.at[r], out_hbm.at[pl.ds(r0 + j * CH, CH)], wsem[r])

        for d in range(min(DEPTH, n_ch)):
            g_copy(d).start()
        for j in range(n_ch):
            g_copy(j).wait()
            w_copy(j).start()
            if j + DEPTH < n_ch:
                if j >= DEPTH:
                    w_copy(j - DEPTH).wait()
                g_copy(j + DEPTH).start()
        for j in range(max(n_ch - 2 * DEPTH, min(DEPTH, n_ch)), n_ch):
            w_copy(j).wait()

    return gather_kernel(idx3d, table)


def _make_lstm_body(ngc, aliased):
    def lstm_body(emb_ref, h0_ref, c0_ref, wt_ref, ut_ref, b_ref,
                  out_ref, ht_ref, ct_ref, h_s, c_s):
        g = pl.program_id(0)

        @pl.when(g == 0)
        def _():
            h_s[...] = h0_ref[...]
            c_s[...] = c0_ref[...]

        ut = ut_ref[...]
        wt = wt_ref[...]
        bb = b_ref[...]
        b2 = B // 2
        hs = [h_s[:, 0:b2], h_s[:, b2:B]]
        cs = [c_s[:, 0:b2], c_s[:, b2:B]]
        nt = (((1,), (1,)), ((), ()))
        for j in range(TSUB):
            for p in range(2):
                x = emb_ref[j, p * b2:(p + 1) * b2, :]
                z = (lax.dot_general(wt, x, nt,
                                     preferred_element_type=jnp.float32)
                     + jnp.dot(ut, hs[p], preferred_element_type=jnp.float32)
                     + bb)
                s_if = jax.nn.sigmoid(z[:2 * H, :])
                i = s_if[:H, :]
                f = s_if[H:, :]
                gg = jnp.tanh(z[2 * H:3 * H, :])
                o = jax.nn.sigmoid(z[3 * H:, :])
                cs[p] = f * cs[p] + i * gg
                hs[p] = o * jnp.tanh(cs[p])
                out_ref[j * H:(j + 1) * H, p * b2:(p + 1) * b2] = hs[p]
        h_s[:, 0:b2] = hs[0]
        h_s[:, b2:B] = hs[1]
        c_s[:, 0:b2] = cs[0]
        c_s[:, b2:B] = cs[1]

        @pl.when(g == ngc - 1)
        def _():
            ht_ref[:, 0:b2] = hs[0]
            ht_ref[:, b2:B] = hs[1]
            ct_ref[:, 0:b2] = cs[0]
            ct_ref[:, b2:B] = cs[1]

    if not aliased:
        return lstm_body

    def lstm_body_aliased(emb_ref, h0_ref, c0_ref, wt_ref, ut_ref, b_ref,
                          obuf_ref, out_ref, ht_ref, ct_ref, h_s, c_s):
        del obuf_ref
        return lstm_body(emb_ref, h0_ref, c0_ref, wt_ref, ut_ref, b_ref,
                         out_ref, ht_ref, ct_ref, h_s, c_s)

    return lstm_body_aliased


def _tc_lstm_chunk(t0, tc, emb_k, ht, ct, wt, ut, b2d, out_buf):
    ngc = tc // TSUB
    in_specs = [
        pl.BlockSpec((TSUB, B, D), lambda g: (g, 0, 0)),
        pl.BlockSpec((H, B), lambda g: (0, 0)),
        pl.BlockSpec((H, B), lambda g: (0, 0)),
        pl.BlockSpec((4 * H, D), lambda g: (0, 0)),
        pl.BlockSpec((4 * H, H), lambda g: (0, 0)),
        pl.BlockSpec((4 * H, 1), lambda g: (0, 0)),
    ]
    operands = [emb_k, ht, ct, wt, ut, b2d]
    aliases = {}
    if out_buf is not None:
        in_specs.append(pl.BlockSpec(memory_space=pl.ANY))
        operands.append(out_buf)
        aliases = {6: 0}
    g0 = t0 // TSUB
    return pl.pallas_call(
        _make_lstm_body(ngc, out_buf is not None),
        grid=(ngc,),
        in_specs=in_specs,
        out_specs=[
            pl.BlockSpec((TSUB * H, B), lambda g: (g0 + g, 0)),
            pl.BlockSpec((H, B), lambda g: (0, 0)),
            pl.BlockSpec((H, B), lambda g: (0, 0)),
        ],
        out_shape=[
            jax.ShapeDtypeStruct((T * H, B), jnp.float32),
            jax.ShapeDtypeStruct((H, B), jnp.float32),
            jax.ShapeDtypeStruct((H, B), jnp.float32),
        ],
        scratch_shapes=[
            pltpu.VMEM((H, B), jnp.float32),
            pltpu.VMEM((H, B), jnp.float32),
        ],
        input_output_aliases=aliases,
    )(*operands)


def kernel(sequence, states_1, states_2, table, W, U, b):
    wt = jnp.transpose(W)
    ut = jnp.transpose(U)
    bt = b.reshape(4 * H, 1)
    ht = jnp.transpose(states_1)
    ct = jnp.transpose(states_2)
    out_buf = None
    seq_t = jnp.transpose(sequence)
    t0 = 0
    for tc in CHUNKS:
        idx3d = lax.slice_in_dim(seq_t, t0, t0 + tc, axis=0
                                 ).reshape(32, tc * B // (32 * CH), CH)
        emb_k = _sc_gather_chunk(idx3d, table, tc).reshape(tc, B, D)
        out_buf, ht, ct = _tc_lstm_chunk(t0, tc, emb_k, ht, ct, wt, ut, bt,
                                         out_buf)
        t0 += tc
    out = jnp.transpose(out_buf.reshape(T, H, B), (2, 0, 1))
    return out, jnp.transpose(ht), jnp.transpose(ct)
